# Initial kernel scaffold; baseline (speedup 1.0000x reference)
import functools

import jax
import jax.numpy as jnp
from jax import lax
from jax.experimental import pallas as pl
from jax.experimental.pallas import tpu as pltpu
from jax.experimental.pallas import tpu_sc as plsc

B, N, M, K = 4, 16384, 1024, 64
L = 16
NCHUNK = N // L
ROW_PAD = L


def _ball_query_body(xyz_hbm, ct_hbm, r2_hbm, out_hbm,
                     xs, ys, zs, cb, r2s, ob):
    nc = 2
    wid = lax.axis_index("s") * nc + lax.axis_index("c")
    nworkers = 32
    cpw = (B * M) // nworkers
    wpb = M // cpw
    b = wid // wpb
    m0 = (wid % wpb) * cpw

    pltpu.sync_copy(xyz_hbm.at[b, 0], xs)
    pltpu.sync_copy(xyz_hbm.at[b, 1], ys)
    pltpu.sync_copy(xyz_hbm.at[b, 2], zs)
    pltpu.sync_copy(ct_hbm.at[b, 0, pl.ds(m0, cpw)], cb.at[0])
    pltpu.sync_copy(ct_hbm.at[b, 1, pl.ds(m0, cpw)], cb.at[1])
    pltpu.sync_copy(ct_hbm.at[b, 2, pl.ds(m0, cpw)], cb.at[2])
    pltpu.sync_copy(r2_hbm, r2s)

    r2v = r2s[...]
    iota = lax.iota(jnp.int32, L)
    zeros = jnp.zeros((L,), jnp.int32)

    def per_center(m, _):
        row = m * K
        for v in range(K // L):
            ob[pl.ds(row + v * L, L)] = zeros

        cxv = jnp.full((L,), cb[0, m], jnp.float32)
        cyv = jnp.full((L,), cb[1, m], jnp.float32)
        czv = jnp.full((L,), cb[2, m], jnp.float32)

        def cond(carry):
            j, cnt = carry
            return jnp.logical_and(j < NCHUNK, cnt < K)

        def body(carry):
            j, cnt = carry
            base = j * L
            dx = xs[pl.ds(base, L)] - cxv
            dy = ys[pl.ds(base, L)] - cyv
            dz = zs[pl.ds(base, L)] - czv
            d2 = dx * dx + dy * dy + dz * dz
            hit = d2 < r2v
            idxv = iota + base
            plsc.store_compressed(ob.at[pl.ds(row + cnt, L)], idxv, hit)
            nhits = jnp.sum(hit.astype(jnp.int32))
            return j + 1, cnt + nhits

        lax.while_loop(cond, body, (jnp.int32(0), jnp.int32(0)))
        return 0

    lax.fori_loop(0, cpw, per_center, 0)
    pltpu.sync_copy(ob.at[pl.ds(0, cpw * K)],
                    out_hbm.at[b, pl.ds(m0 * K, cpw * K)])


def kernel(xyz, center_xyz, max_radius, sample_num):
    xyz_t = jnp.transpose(xyz, (0, 2, 1))
    ct_t = jnp.transpose(center_xyz, (0, 2, 1))
    r2 = jnp.asarray(max_radius, jnp.float32) ** 2
    r2v = jnp.broadcast_to(r2, (L,))

    mesh = plsc.VectorSubcoreMesh(core_axis_name="c", subcore_axis_name="s")
    run = functools.partial(
        pl.kernel,
        mesh=mesh,
        out_type=jax.ShapeDtypeStruct((B, M * K), jnp.int32),
        scratch_types=[
            pltpu.VMEM((N,), jnp.float32),
            pltpu.VMEM((N,), jnp.float32),
            pltpu.VMEM((N,), jnp.float32),
            pltpu.VMEM((3, (B * M) // 32), jnp.float32),
            pltpu.VMEM((L,), jnp.float32),
            pltpu.VMEM(((B * M) // 32 * K + ROW_PAD,), jnp.int32),
        ],
    )(_ball_query_body)
    idx = run(xyz_t, ct_t, r2v).reshape(B, M, K)
    col = lax.broadcasted_iota(jnp.int32, (1, 1, K), 2)
    return jnp.where(col < jnp.asarray(sample_num, jnp.int32), idx, 0)


# speedup vs baseline: 32.5415x; 32.5415x over previous
import functools

import jax
import jax.numpy as jnp
from jax import lax
from jax.experimental import pallas as pl
from jax.experimental.pallas import tpu as pltpu
from jax.experimental.pallas import tpu_sc as plsc

B, N, M, K = 4, 16384, 1024, 64
L = 16
NCHUNK = N // L
ROW_PAD = L


def _ball_query_body(xyz_hbm, ct_hbm, r2_hbm, out_hbm,
                     xs, ys, zs, x2s, cb, r2s, ob):
    nc = 2
    wid = lax.axis_index("s") * nc + lax.axis_index("c")
    nworkers = 32
    cpw = (B * M) // nworkers
    wpb = M // cpw
    b = wid // wpb
    m0 = (wid % wpb) * cpw

    pltpu.sync_copy(xyz_hbm.at[pl.ds((b * 3 + 0) * N, N)], xs)
    pltpu.sync_copy(xyz_hbm.at[pl.ds((b * 3 + 1) * N, N)], ys)
    pltpu.sync_copy(xyz_hbm.at[pl.ds((b * 3 + 2) * N, N)], zs)
    pltpu.sync_copy(ct_hbm.at[pl.ds((b * 3 + 0) * M + m0, cpw)], cb.at[0])
    pltpu.sync_copy(ct_hbm.at[pl.ds((b * 3 + 1) * M + m0, cpw)], cb.at[1])
    pltpu.sync_copy(ct_hbm.at[pl.ds((b * 3 + 2) * M + m0, cpw)], cb.at[2])
    pltpu.sync_copy(r2_hbm, r2s)

    r2v = r2s[...]
    iota = lax.iota(jnp.int32, L)
    zeros = jnp.zeros((L,), jnp.int32)

    def bf16_round(v):
        u = plsc.bitcast(v, jnp.int32)
        lsb = lax.shift_right_logical(u, 16) & 1
        u = (u + (lsb + 0x7FFF)) & jnp.int32(-65536)
        return plsc.bitcast(u, jnp.float32)

    def pre(j, _):
        base = j * L
        px = xs[pl.ds(base, L)]
        py = ys[pl.ds(base, L)]
        pz = zs[pl.ds(base, L)]
        x2s[pl.ds(base, L)] = (px * px + py * py) + pz * pz
        xs[pl.ds(base, L)] = bf16_round(px)
        ys[pl.ds(base, L)] = bf16_round(py)
        zs[pl.ds(base, L)] = bf16_round(pz)
        return 0

    lax.fori_loop(0, NCHUNK, pre, 0)

    def per_group(g, _):
        cx16 = cb[0, pl.ds(g * L, L)]
        cy16 = cb[1, pl.ds(g * L, L)]
        cz16 = cb[2, pl.ds(g * L, L)]
        c216 = (cx16 * cx16 + cy16 * cy16) + cz16 * cz16
        cxb16 = bf16_round(cx16)
        cyb16 = bf16_round(cy16)
        czb16 = bf16_round(cz16)
        for lane in range(L):
            row = (g * L + lane) * K
            for v in range(K // L):
                ob[pl.ds(row + v * L, L)] = zeros

            cxv = jnp.full((L,), cxb16[lane], jnp.float32)
            cyv = jnp.full((L,), cyb16[lane], jnp.float32)
            czv = jnp.full((L,), czb16[lane], jnp.float32)
            c2v = jnp.full((L,), c216[lane], jnp.float32)

            def cond(carry):
                j, cnt = carry
                return jnp.logical_and(j < NCHUNK, cnt < K)

            def body(carry, row=row, cxv=cxv, cyv=cyv, czv=czv, c2v=c2v):
                j, cnt = carry
                base = j * L
                cross = (cxv * xs[pl.ds(base, L)]
                         + cyv * ys[pl.ds(base, L)]) + czv * zs[pl.ds(base, L)]
                d2 = (c2v + x2s[pl.ds(base, L)]) - (cross + cross)
                hit = d2 < r2v
                idxv = iota + base
                plsc.store_compressed(ob.at[pl.ds(row + cnt, L)], idxv,
                                      mask=hit)
                nhits = jnp.sum(hit.astype(jnp.int32))
                return j + 1, cnt + nhits

            lax.while_loop(cond, body, (jnp.int32(0), jnp.int32(0)))
        return 0

    lax.fori_loop(0, cpw // L, per_group, 0)
    pltpu.sync_copy(ob.at[pl.ds(0, cpw * K)],
                    out_hbm.at[pl.ds((b * M + m0) * K, cpw * K)])


def kernel(xyz, center_xyz, max_radius, sample_num):
    xyz_t = jnp.transpose(xyz, (0, 2, 1)).reshape(-1)
    ct_t = jnp.transpose(center_xyz, (0, 2, 1)).reshape(-1)
    r2 = jnp.asarray(max_radius, jnp.float32) ** 2
    r2v = jnp.broadcast_to(r2, (L,))

    mesh = plsc.VectorSubcoreMesh(core_axis_name="c", subcore_axis_name="s")
    run = functools.partial(
        pl.kernel,
        mesh=mesh,
        out_type=jax.ShapeDtypeStruct((B * M * K,), jnp.int32),
        scratch_types=[
            pltpu.VMEM((N,), jnp.float32),
            pltpu.VMEM((N,), jnp.float32),
            pltpu.VMEM((N,), jnp.float32),
            pltpu.VMEM((N,), jnp.float32),
            pltpu.VMEM((3, (B * M) // 32), jnp.float32),
            pltpu.VMEM((L,), jnp.float32),
            pltpu.VMEM(((B * M) // 32 * K + ROW_PAD,), jnp.int32),
        ],
        compiler_params=pltpu.CompilerParams(needs_layout_passes=False),
    )(_ball_query_body)
    idx = run(xyz_t, ct_t, r2v).reshape(B, M, K)
    col = lax.broadcasted_iota(jnp.int32, (1, 1, K), 2)
    return jnp.where(col < jnp.asarray(sample_num, jnp.int32), idx, 0)


# speedup vs baseline: 54.6391x; 1.6791x over previous
import functools

import jax
import jax.numpy as jnp
from jax import lax
from jax.experimental import pallas as pl
from jax.experimental.pallas import tpu as pltpu
from jax.experimental.pallas import tpu_sc as plsc

B, N, M, K = 4, 16384, 1024, 64
L = 16
NCHUNK = N // L
UNROLL = 4
ROW_PAD = UNROLL * L + L


def _ball_query_body(xyz_hbm, ct_hbm, r2_hbm, out_hbm,
                     xs, ys, zs, x2s, cb, r2s, ob):
    nc = 2
    wid = lax.axis_index("s") * nc + lax.axis_index("c")
    nworkers = 32
    cpw = (B * M) // nworkers
    wpb = M // cpw
    b = wid // wpb
    m0 = (wid % wpb) * cpw

    pltpu.sync_copy(xyz_hbm.at[pl.ds((b * 3 + 0) * N, N)], xs)
    pltpu.sync_copy(xyz_hbm.at[pl.ds((b * 3 + 1) * N, N)], ys)
    pltpu.sync_copy(xyz_hbm.at[pl.ds((b * 3 + 2) * N, N)], zs)
    pltpu.sync_copy(ct_hbm.at[pl.ds((b * 3 + 0) * M + m0, cpw)], cb.at[0])
    pltpu.sync_copy(ct_hbm.at[pl.ds((b * 3 + 1) * M + m0, cpw)], cb.at[1])
    pltpu.sync_copy(ct_hbm.at[pl.ds((b * 3 + 2) * M + m0, cpw)], cb.at[2])
    pltpu.sync_copy(r2_hbm, r2s)

    r2v = r2s[...]
    iota = lax.iota(jnp.int32, L)
    zeros = jnp.zeros((L,), jnp.int32)

    def bf16_round(v):
        u = plsc.bitcast(v, jnp.int32)
        lsb = lax.shift_right_logical(u, 16) & 1
        u = (u + (lsb + 0x7FFF)) & jnp.int32(-65536)
        return plsc.bitcast(u, jnp.float32)

    def pre(j, _):
        base = j * L
        px = xs[pl.ds(base, L)]
        py = ys[pl.ds(base, L)]
        pz = zs[pl.ds(base, L)]
        x2s[pl.ds(base, L)] = (px * px + py * py) + pz * pz
        xs[pl.ds(base, L)] = bf16_round(px)
        ys[pl.ds(base, L)] = bf16_round(py)
        zs[pl.ds(base, L)] = bf16_round(pz)
        return 0

    lax.fori_loop(0, NCHUNK, pre, 0)

    def per_group(g, _):
        cx16 = cb[0, pl.ds(g * L, L)]
        cy16 = cb[1, pl.ds(g * L, L)]
        cz16 = cb[2, pl.ds(g * L, L)]
        c216 = (cx16 * cx16 + cy16 * cy16) + cz16 * cz16
        cxb16 = bf16_round(cx16)
        cyb16 = bf16_round(cy16)
        czb16 = bf16_round(cz16)
        for lane in range(L):
            row = (g * L + lane) * K
            for v in range(K // L):
                ob[pl.ds(row + v * L, L)] = zeros

            cxv = jnp.full((L,), cxb16[lane], jnp.float32)
            cyv = jnp.full((L,), cyb16[lane], jnp.float32)
            czv = jnp.full((L,), czb16[lane], jnp.float32)
            c2v = jnp.full((L,), c216[lane], jnp.float32)

            def cond(carry):
                j, cnt = carry
                return jnp.logical_and(j < NCHUNK, cnt < K)

            def body(carry, row=row, cxv=cxv, cyv=cyv, czv=czv, c2v=c2v):
                j, cnt = carry
                for u in range(UNROLL):
                    base = (j + u) * L
                    cross = (cxv * xs[pl.ds(base, L)]
                             + cyv * ys[pl.ds(base, L)]
                             ) + czv * zs[pl.ds(base, L)]
                    d2 = (c2v + x2s[pl.ds(base, L)]) - (cross + cross)
                    hit = d2 < r2v
                    idxv = iota + base
                    plsc.store_compressed(ob.at[pl.ds(row + cnt, L)], idxv,
                                          mask=hit)
                    cnt = cnt + plsc.all_reduce_population_count(hit)[0]
                return j + UNROLL, cnt

            lax.while_loop(cond, body, (jnp.int32(0), jnp.int32(0)))
        return 0

    lax.fori_loop(0, cpw // L, per_group, 0)
    pltpu.sync_copy(ob.at[pl.ds(0, cpw * K)],
                    out_hbm.at[pl.ds((b * M + m0) * K, cpw * K)])


def kernel(xyz, center_xyz, max_radius, sample_num):
    xyz_t = jnp.transpose(xyz, (0, 2, 1)).reshape(-1)
    ct_t = jnp.transpose(center_xyz, (0, 2, 1)).reshape(-1)
    r2 = jnp.asarray(max_radius, jnp.float32) ** 2
    r2v = jnp.broadcast_to(r2, (L,))

    mesh = plsc.VectorSubcoreMesh(core_axis_name="c", subcore_axis_name="s")
    run = functools.partial(
        pl.kernel,
        mesh=mesh,
        out_type=jax.ShapeDtypeStruct((B * M * K,), jnp.int32),
        scratch_types=[
            pltpu.VMEM((N,), jnp.float32),
            pltpu.VMEM((N,), jnp.float32),
            pltpu.VMEM((N,), jnp.float32),
            pltpu.VMEM((N,), jnp.float32),
            pltpu.VMEM((3, (B * M) // 32), jnp.float32),
            pltpu.VMEM((L,), jnp.float32),
            pltpu.VMEM(((B * M) // 32 * K + ROW_PAD,), jnp.int32),
        ],
        compiler_params=pltpu.CompilerParams(needs_layout_passes=False),
    )(_ball_query_body)
    idx = run(xyz_t, ct_t, r2v).reshape(B, M, K)
    col = lax.broadcasted_iota(jnp.int32, (1, 1, K), 2)
    return jnp.where(col < jnp.asarray(sample_num, jnp.int32), idx, 0)
